# Initial kernel scaffold; baseline (speedup 1.0000x reference)
import jax
import jax.numpy as jnp
from jax.experimental import pallas as pl


def kernel(edge_index, edge_values, user_emb, item_emb, gc_w0, gc_b0, gc_w1, gc_b1, gc_w2, gc_b2, bi_w0, bi_b0, bi_w1, bi_b1, bi_w2, bi_b2):
    raise NotImplementedError("write your pallas kernel here")


# speedup vs baseline: 6.3863x; 6.3863x over previous
import functools

import jax
import jax.numpy as jnp
from jax import lax
from jax.experimental import pallas as pl
from jax.experimental.pallas import tpu as pltpu
from jax.experimental.pallas import tpu_sc as plsc

NUM_USERS = 50000
NUM_ITEMS = 50000
N = NUM_USERS + NUM_ITEMS
E = 1600000
D = 32
H = 16
N_LAYERS = 3

NC, NS, L = 2, 16, 16
GB = 128
CHUNK_ROWS = 8
CHUNK_E = CHUNK_ROWS * GB
ROWS_PS = 800
NCHUNK = ROWS_PS // CHUNK_ROWS
EPS = ROWS_PS * GB
E_PAD = EPS * NS
RPS = 6256
N_ACC = RPS * NS

_mesh = plsc.VectorSubcoreMesh(core_axis_name="c", subcore_axis_name="s")


@functools.partial(
    pl.kernel,
    out_type=jax.ShapeDtypeStruct((2 * N_ACC, H), jnp.float32),
    mesh=_mesh,
    compiler_params=pltpu.CompilerParams(use_tc_tiling_on_sc=False),
    scratch_types=[
        pltpu.VMEM((CHUNK_E, H), jnp.float32),
        pltpu.VMEM((CHUNK_E,), jnp.float32),
        pltpu.VMEM((CHUNK_E,), jnp.int32),
        pltpu.VMEM((CHUNK_E,), jnp.int32),
        pltpu.VMEM((CHUNK_ROWS, GB), jnp.int32),
        pltpu.VMEM_SHARED((N_ACC, H), jnp.float32),
        pltpu.SemaphoreType.DMA,
        pltpu.SemaphoreType.DMA,
    ],
)
def _sc_spmv(tbl_hbm, src_hbm, dst_hbm, ev_hbm, zero_hbm, out_hbm,
             rbuf, evbuf, srcbuf, slbuf, dstbuf, acc, gsem, ssem):
    c = lax.axis_index("c")
    s = lax.axis_index("s")
    pltpu.sync_copy(zero_hbm.at[pl.ds(s * RPS, RPS)], acc.at[pl.ds(s * RPS, RPS)])
    plsc.subcore_barrier()

    coff = jnp.full((L,), c * N, jnp.int32)
    base_row = s * ROWS_PS

    @pl.loop(0, NCHUNK)
    def _chunk(k):
        row0 = base_row + k * CHUNK_ROWS
        e0 = row0 * GB
        pltpu.sync_copy(src_hbm.at[pl.ds(e0, CHUNK_E)], srcbuf)
        pltpu.sync_copy(ev_hbm.at[pl.ds(e0, CHUNK_E)], evbuf)
        pltpu.sync_copy(dst_hbm.at[pl.ds(row0, CHUNK_ROWS)], dstbuf)

        @pl.loop(0, CHUNK_E, step=L)
        def _off(i):
            slbuf[pl.ds(i, L)] = srcbuf[pl.ds(i, L)] + coff

        gathers = [
            pltpu.async_copy(
                tbl_hbm.at[slbuf.at[pl.ds(j * GB, GB)]],
                rbuf.at[pl.ds(j * GB, GB)], gsem)
            for j in range(CHUNK_ROWS)
        ]
        for g in gathers:
            g.wait()

        @pl.loop(0, CHUNK_E, step=L)
        def _scale(b):
            vv = evbuf[pl.ds(b, L)]
            for e in range(L):
                rbuf[b + e, :] = rbuf[b + e, :] * vv[e]

        scatters = [
            pltpu.async_copy(
                rbuf.at[pl.ds(j * GB, GB)],
                acc.at[dstbuf.at[j]], ssem, add=True)
            for j in range(CHUNK_ROWS)
        ]
        for sc in scatters:
            sc.wait()

    plsc.subcore_barrier()
    pltpu.sync_copy(acc.at[pl.ds(s * RPS, RPS)],
                    out_hbm.at[pl.ds(c * N_ACC + s * RPS, RPS)])


_TC_R = 1000


def _tc_body(s_ref, ego_ref, w1_ref, b1_ref, w2_ref, b2_ref,
             ego_o, tbl_o, nrm_o):
    side = jnp.concatenate([s_ref[0], s_ref[1]], axis=1)
    ego = ego_ref[...]
    sm = jnp.dot(side, w1_ref[...], precision=lax.Precision.HIGHEST,
                 preferred_element_type=jnp.float32) + b1_ref[...]
    sm = jnp.where(sm >= 0, sm, 0.01 * sm)
    bi = jnp.dot(ego * side, w2_ref[...], precision=lax.Precision.HIGHEST,
                 preferred_element_type=jnp.float32) + b2_ref[...]
    bi = jnp.where(bi >= 0, bi, 0.01 * bi)
    e2 = sm + bi
    ego_o[...] = e2
    tbl_o[0] = e2[:, :H]
    tbl_o[1] = e2[:, H:]
    nz = jnp.sum(e2 * e2, axis=1, keepdims=True)
    nrm_o[...] = e2 / jnp.maximum(jnp.sqrt(nz), 1e-12)


def _tc_dense(side_tbl, ego, w1, b1, w2, b2):
    grid = (N // _TC_R,)
    ego_next, tbl, nrm = pl.pallas_call(
        _tc_body,
        grid=grid,
        in_specs=[
            pl.BlockSpec((2, _TC_R, H), lambda i: (0, i, 0)),
            pl.BlockSpec((_TC_R, D), lambda i: (i, 0)),
            pl.BlockSpec((D, D), lambda i: (0, 0)),
            pl.BlockSpec((1, D), lambda i: (0, 0)),
            pl.BlockSpec((D, D), lambda i: (0, 0)),
            pl.BlockSpec((1, D), lambda i: (0, 0)),
        ],
        out_specs=[
            pl.BlockSpec((_TC_R, D), lambda i: (i, 0)),
            pl.BlockSpec((2, _TC_R, H), lambda i: (0, i, 0)),
            pl.BlockSpec((_TC_R, D), lambda i: (i, 0)),
        ],
        out_shape=[
            jax.ShapeDtypeStruct((N, D), jnp.float32),
            jax.ShapeDtypeStruct((2, N, H), jnp.float32),
            jax.ShapeDtypeStruct((N, D), jnp.float32),
        ],
    )(side_tbl.reshape(2, N_ACC, H), ego, w1, b1.reshape(1, D), w2, b2.reshape(1, D))
    return ego_next, tbl.reshape(2 * N, H), nrm


def kernel(edge_index, edge_values, user_emb, item_emb,
           gc_w0, gc_b0, gc_w1, gc_b1, gc_w2, gc_b2,
           bi_w0, bi_b0, bi_w1, bi_b1, bi_w2, bi_b2):
    gc = [(gc_w0, gc_b0), (gc_w1, gc_b1), (gc_w2, gc_b2)]
    bi = [(bi_w0, bi_b0), (bi_w1, bi_b1), (bi_w2, bi_b2)]

    pad = E_PAD - E
    src = jnp.concatenate(
        [edge_index[0].astype(jnp.int32), jnp.zeros((pad,), jnp.int32)])
    dst2d = jnp.concatenate(
        [edge_index[1].astype(jnp.int32), jnp.zeros((pad,), jnp.int32)]
    ).reshape(E_PAD // GB, GB)
    ev = jnp.concatenate([edge_values, jnp.zeros((pad,), jnp.float32)])
    zeros = jnp.zeros((N_ACC, H), jnp.float32)

    ego = jnp.concatenate([user_emb, item_emb], axis=0)
    tbl = jnp.concatenate([ego[:, :H], ego[:, H:]], axis=0)

    outs = [ego]
    for i in range(N_LAYERS):
        side_tbl = _sc_spmv(tbl, src, dst2d, ev, zeros)
        ego, tbl, nrm = _tc_dense(side_tbl, ego, gc[i][0], gc[i][1],
                                  bi[i][0], bi[i][1])
        outs.append(nrm)

    all_emb = jnp.concatenate(outs, axis=1)
    return all_emb[:NUM_USERS], all_emb[NUM_USERS:]


# speedup vs baseline: 7.9403x; 1.2433x over previous
import functools

import jax
import jax.numpy as jnp
from jax import lax
from jax.experimental import pallas as pl
from jax.experimental.pallas import tpu as pltpu
from jax.experimental.pallas import tpu_sc as plsc

NUM_USERS = 50000
NUM_ITEMS = 50000
N = NUM_USERS + NUM_ITEMS
E = 1600000
D = 32
H = 16
N_LAYERS = 3

NC, NS, L = 2, 16, 16
GB = 128
CHUNK_ROWS = 4
CHUNK_E = CHUNK_ROWS * GB
SB = 8
SB_E = SB * CHUNK_E
SB_ROWS = SB * CHUNK_ROWS
ROWS_PS = 800
EPS = ROWS_PS * GB
NCHUNK = EPS // CHUNK_E
NSB = NCHUNK // SB
E_PAD = EPS * NS
RPS = 6256
N_ACC = RPS * NS

_mesh = plsc.VectorSubcoreMesh(core_axis_name="c", subcore_axis_name="s")


@functools.partial(
    pl.kernel,
    out_type=jax.ShapeDtypeStruct((2 * N_ACC, H), jnp.float32),
    mesh=_mesh,
    compiler_params=pltpu.CompilerParams(use_tc_tiling_on_sc=False),
    scratch_types=[
        pltpu.VMEM((2, CHUNK_E, H), jnp.float32),
        pltpu.VMEM((SB_E,), jnp.float32),
        pltpu.VMEM((SB_E,), jnp.int32),
        pltpu.VMEM((SB_ROWS, GB), jnp.int32),
        pltpu.VMEM_SHARED((N_ACC, H), jnp.float32),
        pltpu.SemaphoreType.DMA,
        pltpu.SemaphoreType.DMA,
        pltpu.SemaphoreType.DMA,
        pltpu.SemaphoreType.DMA,
    ],
)
def _sc_spmv(tbl_hbm, src_hbm, dst_hbm, ev_hbm, zero_hbm, out_hbm,
             rbuf, evbuf, slbuf, dstbuf, acc, g0, g1, s0, s1):
    c = lax.axis_index("c")
    s = lax.axis_index("s")
    gsem = [g0, g1]
    ssem = [s0, s1]
    pltpu.sync_copy(zero_hbm.at[pl.ds(s * RPS, RPS)], acc.at[pl.ds(s * RPS, RPS)])
    plsc.subcore_barrier()

    base_row = s * ROWS_PS

    def fire_gathers(j, p):
        return [
            pltpu.async_copy(
                tbl_hbm.at[slbuf.at[pl.ds(j * CHUNK_E + t * GB, GB)]],
                rbuf.at[p, pl.ds(t * GB, GB)], gsem[p])
            for t in range(CHUNK_ROWS)
        ]

    def fire_scatters(j, p):
        return [
            pltpu.async_copy(
                rbuf.at[p, pl.ds(t * GB, GB)],
                acc.at[dstbuf.at[j * CHUNK_ROWS + t]], ssem[p], add=True)
            for t in range(CHUNK_ROWS)
        ]

    @pl.loop(0, NSB)
    def _sb(q):
        row0 = base_row + q * SB_ROWS
        e0 = row0 * GB
        pltpu.sync_copy(src_hbm.at[c].at[pl.ds(e0, SB_E)], slbuf)
        pltpu.sync_copy(ev_hbm.at[pl.ds(e0, SB_E)], evbuf)
        pltpu.sync_copy(dst_hbm.at[pl.ds(row0, SB_ROWS)], dstbuf)

        gh = {0: fire_gathers(0, 0)}
        sh = {}
        for j in range(SB):
            p = j & 1
            for h in gh.pop(j):
                h.wait()
            if j + 1 < SB:
                if j - 1 in sh:
                    for h in sh.pop(j - 1):
                        h.wait()
                gh[j + 1] = fire_gathers(j + 1, p ^ 1)

            @pl.loop(0, CHUNK_E, step=L)
            def _scale(b, j=j, p=p):
                vv = evbuf[pl.ds(j * CHUNK_E + b, L)]
                for e in range(L):
                    rbuf[p, b + e, :] = rbuf[p, b + e, :] * vv[e]

            sh[j] = fire_scatters(j, p)
        for j in sorted(sh):
            for h in sh.pop(j):
                h.wait()

    plsc.subcore_barrier()
    pltpu.sync_copy(acc.at[pl.ds(s * RPS, RPS)],
                    out_hbm.at[pl.ds(c * N_ACC + s * RPS, RPS)])


_TC_R = 1000


def _tc_body(s_ref, ego_ref, w1_ref, b1_ref, w2_ref, b2_ref,
             ego_o, tbl_o, nrm_o):
    side = jnp.concatenate([s_ref[0], s_ref[1]], axis=1)
    ego = ego_ref[...]
    sm = jnp.dot(side, w1_ref[...], precision=lax.Precision.HIGHEST,
                 preferred_element_type=jnp.float32) + b1_ref[...]
    sm = jnp.where(sm >= 0, sm, 0.01 * sm)
    bi = jnp.dot(ego * side, w2_ref[...], precision=lax.Precision.HIGHEST,
                 preferred_element_type=jnp.float32) + b2_ref[...]
    bi = jnp.where(bi >= 0, bi, 0.01 * bi)
    e2 = sm + bi
    ego_o[...] = e2
    tbl_o[0] = e2[:, :H]
    tbl_o[1] = e2[:, H:]
    nz = jnp.sum(e2 * e2, axis=1, keepdims=True)
    nrm_o[...] = e2 / jnp.maximum(jnp.sqrt(nz), 1e-12)


def _tc_dense(side_tbl, ego, w1, b1, w2, b2):
    grid = (N // _TC_R,)
    ego_next, tbl, nrm = pl.pallas_call(
        _tc_body,
        grid=grid,
        in_specs=[
            pl.BlockSpec((2, _TC_R, H), lambda i: (0, i, 0)),
            pl.BlockSpec((_TC_R, D), lambda i: (i, 0)),
            pl.BlockSpec((D, D), lambda i: (0, 0)),
            pl.BlockSpec((1, D), lambda i: (0, 0)),
            pl.BlockSpec((D, D), lambda i: (0, 0)),
            pl.BlockSpec((1, D), lambda i: (0, 0)),
        ],
        out_specs=[
            pl.BlockSpec((_TC_R, D), lambda i: (i, 0)),
            pl.BlockSpec((2, _TC_R, H), lambda i: (0, i, 0)),
            pl.BlockSpec((_TC_R, D), lambda i: (i, 0)),
        ],
        out_shape=[
            jax.ShapeDtypeStruct((N, D), jnp.float32),
            jax.ShapeDtypeStruct((2, N, H), jnp.float32),
            jax.ShapeDtypeStruct((N, D), jnp.float32),
        ],
    )(side_tbl.reshape(2, N_ACC, H), ego, w1, b1.reshape(1, D), w2, b2.reshape(1, D))
    return ego_next, tbl.reshape(2 * N, H), nrm


def kernel(edge_index, edge_values, user_emb, item_emb,
           gc_w0, gc_b0, gc_w1, gc_b1, gc_w2, gc_b2,
           bi_w0, bi_b0, bi_w1, bi_b1, bi_w2, bi_b2):
    gc = [(gc_w0, gc_b0), (gc_w1, gc_b1), (gc_w2, gc_b2)]
    bi = [(bi_w0, bi_b0), (bi_w1, bi_b1), (bi_w2, bi_b2)]

    pad = E_PAD - E
    src = jnp.concatenate(
        [edge_index[0].astype(jnp.int32), jnp.zeros((pad,), jnp.int32)])
    src = jnp.stack([src, src + N])
    dst2d = jnp.concatenate(
        [edge_index[1].astype(jnp.int32), jnp.zeros((pad,), jnp.int32)]
    ).reshape(E_PAD // GB, GB)
    ev = jnp.concatenate([edge_values, jnp.zeros((pad,), jnp.float32)])
    zeros = jnp.zeros((N_ACC, H), jnp.float32)

    ego = jnp.concatenate([user_emb, item_emb], axis=0)
    tbl = jnp.concatenate([ego[:, :H], ego[:, H:]], axis=0)

    outs = [ego]
    for i in range(N_LAYERS):
        side_tbl = _sc_spmv(tbl, src, dst2d, ev, zeros)
        ego, tbl, nrm = _tc_dense(side_tbl, ego, gc[i][0], gc[i][1],
                                  bi[i][0], bi[i][1])
        outs.append(nrm)

    all_emb = jnp.concatenate(outs, axis=1)
    return all_emb[:NUM_USERS], all_emb[NUM_USERS:]
